# baseline (speedup 1.0000x reference)
import functools
import math

import jax
import jax.numpy as jnp
from jax import lax
from jax.experimental import pallas as pl
from jax.experimental.pallas import tpu as pltpu
from jax.experimental.pallas import tpu_sc as plsc

D_MODEL = 64
SCALE = math.sqrt(D_MODEL)
LANES = 16

_info = plsc.get_sparse_core_info()
NUM_CORES = _info.num_cores
NUM_SUBCORES = _info.num_subcores
NW = NUM_CORES * NUM_SUBCORES

B_TOTAL = 4096 * 200
BPW = B_TOTAL // NW
CHUNK = 1024
G = BPW // CHUNK

_mesh = plsc.VectorSubcoreMesh(core_axis_name="c", subcore_axis_name="s")


@functools.partial(
    pl.kernel,
    mesh=_mesh,
    out_type=jax.ShapeDtypeStruct((B_TOTAL, D_MODEL), jnp.float32),
    scratch_types=[
        pltpu.VMEM((CHUNK,), jnp.int32),
        pltpu.VMEM((CHUNK, D_MODEL), jnp.float32),
        pltpu.SemaphoreType.DMA,
    ],
    compiler_params=pltpu.CompilerParams(use_tc_tiling_on_sc=False),
)
def _emb_lookup(x_hbm, table_hbm, out_hbm, idx_v, rows_v, gsem):
    wid = lax.axis_index("s") * NUM_CORES + lax.axis_index("c")
    base = wid * BPW

    def chunk_body(g, carry):
        off = base + g * CHUNK
        pltpu.sync_copy(x_hbm.at[pl.ds(off, CHUNK)], idx_v)
        pltpu.async_copy(table_hbm.at[idx_v], rows_v, gsem).wait()

        def scale_row(j, c2):
            for c in range(D_MODEL // LANES):
                sl = pl.ds(c * LANES, LANES)
                rows_v[j, sl] = rows_v[j, sl] * SCALE
            return c2

        lax.fori_loop(0, CHUNK, scale_row, 0, unroll=4)
        pltpu.sync_copy(rows_v, out_hbm.at[pl.ds(off, CHUNK)])
        return carry

    lax.fori_loop(0, G, chunk_body, 0)


def kernel(x, table):
    xf = x.reshape(-1).astype(jnp.int32)
    out = _emb_lookup(xf, table)
    return out.reshape(x.shape[0], x.shape[1], D_MODEL)
